# baseline (speedup 1.0000x reference)
import functools

import jax
import jax.numpy as jnp
from jax import lax
from jax.experimental import pallas as pl
from jax.experimental.pallas import tpu as pltpu
from jax.experimental.pallas import tpu_sc as plsc

B, L, V, D, H, NB = 4096, 200, 1000000, 64, 256, 4
NC, NS = 2, 16
NW = NC * NS
ROWS_W = B // NW
CHUNK = 100
CPR = L // CHUNK
NCH = ROWS_W * CPR
NLANE = 16
NVEC = D // NLANE


def _sc_gather_mean_body(ids_hbm, table_hbm, out_hbm, idx_v, rows_v, acc_v, sem):
    wid = lax.axis_index("s") * NC + lax.axis_index("c")
    pltpu.sync_copy(ids_hbm.at[wid], idx_v)

    def row_body(r, _):
        cps = [
            pltpu.async_copy(table_hbm.at[idx_v.at[CPR * r + c]], rows_v.at[c], sem)
            for c in range(CPR)
        ]
        for cp in cps:
            cp.wait()

        def acc_body(l, accs):
            accs = list(accs)
            for c in range(CPR):
                for k in range(NVEC):
                    accs[c * NVEC + k] = accs[c * NVEC + k] + rows_v[
                        c, l, pl.ds(k * NLANE, NLANE)
                    ]
            return tuple(accs)

        accs = lax.fori_loop(
            0,
            CHUNK,
            acc_body,
            tuple(jnp.zeros((NLANE,), jnp.float32) for _ in range(CPR * NVEC)),
        )
        scale = jnp.float32(1.0 / L)
        for k in range(NVEC):
            tot = accs[k]
            for c in range(1, CPR):
                tot = tot + accs[c * NVEC + k]
            acc_v[r, pl.ds(k * NLANE, NLANE)] = tot * scale
        return 0

    lax.fori_loop(0, ROWS_W, row_body, 0)
    pltpu.sync_copy(acc_v, out_hbm.at[pl.ds(wid * ROWS_W, ROWS_W)])


def _sc_gather_mean(ids3, table):
    mesh = plsc.VectorSubcoreMesh(core_axis_name="c", subcore_axis_name="s")
    return pl.kernel(
        _sc_gather_mean_body,
        out_type=jax.ShapeDtypeStruct((B, D), jnp.float32),
        mesh=mesh,
        scratch_types=[
            pltpu.VMEM((NCH, CHUNK), jnp.int32),
            pltpu.VMEM((CPR, CHUNK, D), jnp.float32),
            pltpu.VMEM((ROWS_W, D), jnp.float32),
            pltpu.SemaphoreType.DMA,
        ],
        compiler_params=pltpu.CompilerParams(use_tc_tiling_on_sc=False),
    )(ids3, table)


def _mlp_body(cur_ref, ctx_ref, w1a_ref, w1b_ref, b1_ref, w2_ref, b2_ref, out_ref):
    cur = cur_ref[...]
    ctx = ctx_ref[...]
    for i in range(NB):
        h = (
            jnp.dot(cur, w1a_ref[i], preferred_element_type=jnp.float32)
            + jnp.dot(ctx, w1b_ref[i], preferred_element_type=jnp.float32)
            + b1_ref[i][None, :]
        )
        h = jnp.maximum(h, 0.0)
        cur = cur + jnp.dot(h, w2_ref[i], preferred_element_type=jnp.float32) + b2_ref[i][None, :]
    out_ref[...] = cur


def _tc_mlp(cur0, ctx, W1, b1, W2, b2):
    w1a = W1[:, :D, :]
    w1b = W1[:, D:, :]
    bm = 512
    grid = B // bm
    return pl.pallas_call(
        _mlp_body,
        grid=(grid,),
        in_specs=[
            pl.BlockSpec((bm, D), lambda i: (i, 0)),
            pl.BlockSpec((bm, D), lambda i: (i, 0)),
            pl.BlockSpec((NB, D, H), lambda i: (0, 0, 0)),
            pl.BlockSpec((NB, D, H), lambda i: (0, 0, 0)),
            pl.BlockSpec((NB, H), lambda i: (0, 0)),
            pl.BlockSpec((NB, H, D), lambda i: (0, 0, 0)),
            pl.BlockSpec((NB, D), lambda i: (0, 0)),
        ],
        out_specs=pl.BlockSpec((bm, D), lambda i: (i, 0)),
        out_shape=jax.ShapeDtypeStruct((B, D), jnp.float32),
    )(cur0, ctx, w1a, w1b, b1, W2, b2)


def kernel(initial_noisy_embedding, context_ids, embedding_table, W1, b1, W2, b2):
    ids3 = context_ids.astype(jnp.int32).reshape(NW, NCH, CHUNK)
    ctx = _sc_gather_mean(ids3, embedding_table)
    return _tc_mlp(initial_noisy_embedding, ctx, W1, b1, W2, b2)


# speedup vs baseline: 1.1279x; 1.1279x over previous
import functools

import jax
import jax.numpy as jnp
from jax import lax
from jax.experimental import pallas as pl
from jax.experimental.pallas import tpu as pltpu
from jax.experimental.pallas import tpu_sc as plsc

B, L, V, D, H, NB = 4096, 200, 1000000, 64, 256, 4
NC, NS = 2, 16
NW = NC * NS
ROWS_W = B // NW
CHUNK = 100
CPR = L // CHUNK
NCH = ROWS_W * CPR
NLANE = 16
NVEC = D // NLANE


def _sc_gather_mean_body(ids_hbm, table_hbm, out_hbm, idx_v, rows_v, acc_v, sem0, sem1):
    wid = lax.axis_index("s") * NC + lax.axis_index("c")
    pltpu.sync_copy(ids_hbm.at[wid], idx_v)
    sems = (sem0, sem1)

    def fetch(r, p):
        for c in range(CPR):
            pltpu.async_copy(
                table_hbm.at[idx_v.at[CPR * r + c]], rows_v.at[p, c], sems[p]
            )

    for p in range(2):
        fetch(p, p)

    def pair_body(g, _):
        for p in range(2):
            r = 2 * g + p
            for c in range(CPR):
                pltpu.make_async_copy(
                    table_hbm.at[pl.ds(0, CHUNK)], rows_v.at[p, c], sems[p]
                ).wait()

            def acc_body(l, accs):
                accs = list(accs)
                for u in range(2):
                    for c in range(CPR):
                        for k in range(NVEC):
                            accs[c * NVEC + k] = accs[c * NVEC + k] + rows_v[
                                p, c, 2 * l + u, pl.ds(k * NLANE, NLANE)
                            ]
                return tuple(accs)

            accs = lax.fori_loop(
                0,
                CHUNK // 2,
                acc_body,
                tuple(jnp.zeros((NLANE,), jnp.float32) for _ in range(CPR * NVEC)),
            )
            scale = jnp.float32(1.0 / L)
            for k in range(NVEC):
                tot = accs[k]
                for c in range(1, CPR):
                    tot = tot + accs[c * NVEC + k]
                acc_v[r, pl.ds(k * NLANE, NLANE)] = tot * scale

            nr = r + 2

            @pl.when(nr < ROWS_W)
            def _():
                fetch(nr, p)

        return 0

    lax.fori_loop(0, ROWS_W // 2, pair_body, 0)
    pltpu.sync_copy(acc_v, out_hbm.at[pl.ds(wid * ROWS_W, ROWS_W)])


def _sc_gather_mean(ids3, table):
    mesh = plsc.VectorSubcoreMesh(core_axis_name="c", subcore_axis_name="s")
    return pl.kernel(
        _sc_gather_mean_body,
        out_type=jax.ShapeDtypeStruct((B, D), jnp.float32),
        mesh=mesh,
        scratch_types=[
            pltpu.VMEM((NCH, CHUNK), jnp.int32),
            pltpu.VMEM((2, CPR, CHUNK, D), jnp.float32),
            pltpu.VMEM((ROWS_W, D), jnp.float32),
            pltpu.SemaphoreType.DMA,
            pltpu.SemaphoreType.DMA,
        ],
        compiler_params=pltpu.CompilerParams(use_tc_tiling_on_sc=False),
    )(ids3, table)


def _mlp_body(cur_ref, ctx_ref, w1a_ref, w1b_ref, b1_ref, w2_ref, b2_ref, out_ref):
    cur = cur_ref[...]
    ctx = ctx_ref[...]
    for i in range(NB):
        h = (
            jnp.dot(cur, w1a_ref[i], preferred_element_type=jnp.float32)
            + jnp.dot(ctx, w1b_ref[i], preferred_element_type=jnp.float32)
            + b1_ref[i][None, :]
        )
        h = jnp.maximum(h, 0.0)
        cur = cur + jnp.dot(h, w2_ref[i], preferred_element_type=jnp.float32) + b2_ref[i][None, :]
    out_ref[...] = cur


def _tc_mlp(cur0, ctx, W1, b1, W2, b2):
    w1a = W1[:, :D, :]
    w1b = W1[:, D:, :]
    bm = 512
    grid = B // bm
    return pl.pallas_call(
        _mlp_body,
        grid=(grid,),
        in_specs=[
            pl.BlockSpec((bm, D), lambda i: (i, 0)),
            pl.BlockSpec((bm, D), lambda i: (i, 0)),
            pl.BlockSpec((NB, D, H), lambda i: (0, 0, 0)),
            pl.BlockSpec((NB, D, H), lambda i: (0, 0, 0)),
            pl.BlockSpec((NB, H), lambda i: (0, 0)),
            pl.BlockSpec((NB, H, D), lambda i: (0, 0, 0)),
            pl.BlockSpec((NB, D), lambda i: (0, 0)),
        ],
        out_specs=pl.BlockSpec((bm, D), lambda i: (i, 0)),
        out_shape=jax.ShapeDtypeStruct((B, D), jnp.float32),
    )(cur0, ctx, w1a, w1b, b1, W2, b2)


def kernel(initial_noisy_embedding, context_ids, embedding_table, W1, b1, W2, b2):
    ids3 = context_ids.astype(jnp.int32).reshape(NW, NCH, CHUNK)
    ctx = _sc_gather_mean(ids3, embedding_table)
    return _tc_mlp(initial_noisy_embedding, ctx, W1, b1, W2, b2)
